# baseline (speedup 1.0000x reference)
import jax
import jax.numpy as jnp
from jax.experimental import pallas as pl
from jax.experimental.pallas import tpu as pltpu

NOBJ = 10
NVAL = 50
NPER = 1 + NOBJ + NVAL
NPAD = 64
E = 128
H = 128
NHEADS = 4
DH = H // NHEADS
G = 32
NH4 = NHEADS * NPAD


def _kernel(head_ref, obj_ref, val_ref, edges_ref,
            wh_ref, bh_ref, wo_ref, bo_ref, wv_ref, bv_ref,
            w0_ref, a0_ref, b0_ref, w1_ref, a1_ref, b1_ref,
            outh_ref, outv_ref):
    f32 = jnp.float32
    bf16 = jnp.bfloat16
    xh = jnp.maximum(jnp.dot(head_ref[:], wh_ref[:],
                             preferred_element_type=f32) + bh_ref[:], 0.0)
    xo = jnp.maximum(jnp.dot(obj_ref[:], wo_ref[:],
                             preferred_element_type=f32) + bo_ref[:], 0.0)
    xv = jnp.maximum(jnp.dot(val_ref[:], wv_ref[:],
                             preferred_element_type=f32) + bv_ref[:], 0.0)
    x = jnp.concatenate([
        xh.reshape(G, 1, H), xo.reshape(G, NOBJ, H), xv.reshape(G, NVAL, H),
        jnp.zeros((G, NPAD - NPER, H), f32)], axis=1).reshape(G * NPAD, H)

    ei = edges_ref[:]
    iota_n = jax.lax.broadcasted_iota(jnp.int32, (G, NPAD, E), 1)
    srow = jnp.where(ei[:, 0:1, :] == iota_n, 1.0, 0.0).astype(bf16)
    drow = jnp.where(ei[:, 1:2, :] == iota_n, 1.0, 0.0).astype(bf16)
    cmat = jax.lax.dot_general(drow, srow, (((2,), (2,)), ((0,), (0,))),
                               preferred_element_type=f32)
    eye = jnp.where(
        jax.lax.broadcasted_iota(jnp.int32, (NPAD, NPAD), 0)
        == jax.lax.broadcasted_iota(jnp.int32, (NPAD, NPAD), 1), 1.0, 0.0)
    cmat = cmat + eye[None]
    c4 = jnp.concatenate([cmat] * NHEADS, axis=1)

    i1 = jax.lax.broadcasted_iota(jnp.int32, (G, NH4, 2 * NHEADS), 1) // NPAD
    i2 = jax.lax.broadcasted_iota(jnp.int32, (G, NH4, 2 * NHEADS), 2)
    sel_dst = jnp.where(i1 == i2 - NHEADS, 1.0, 0.0)
    ind_src = jnp.where(i1 == i2, 1.0, 0.0)
    j2 = jax.lax.broadcasted_iota(jnp.int32, (G, NPAD, 2 * NHEADS), 2)
    sel_src = jnp.where(j2 < NHEADS, 1.0, 0.0)
    ones_dst = jnp.where(j2 >= NHEADS, 1.0, 0.0)
    ones_den = jnp.ones((G, NPAD, 8), f32)
    lane = jax.lax.broadcasted_iota(jnp.int32, (1, 1, H), 2) // DH

    def gat_layer(xp2, acat_ref, bias_ref):
        xp3 = xp2.reshape(G, NPAD, H)
        a3 = jnp.dot(xp2, acat_ref[:],
                     preferred_element_type=f32).reshape(G, NPAD, 2 * NHEADS)
        a8t = jnp.concatenate([a3] * NHEADS, axis=1)
        x8 = a8t * sel_dst + ind_src
        y8 = a3 * sel_src + ones_dst
        lg = jax.lax.dot_general(x8, y8, (((2,), (2,)), ((0,), (0,))),
                                 preferred_element_type=f32)
        lg = jnp.maximum(lg, 0.2 * lg)
        ex = c4 * jnp.exp(lg)
        den = jax.lax.dot_general(ex, ones_den, (((2,), (1,)), ((0,), (0,))),
                                  preferred_element_type=f32)[:, :, 0:1]
        att = ex * (1.0 / den)
        of = jax.lax.dot_general(att, xp3, (((2,), (1,)), ((0,), (0,))),
                                 preferred_element_type=f32)
        out = jnp.where(lane == 0, of[:, 0 * NPAD:1 * NPAD, :], 0.0)
        for h in range(1, NHEADS):
            out = out + jnp.where(lane == h,
                                  of[:, h * NPAD:(h + 1) * NPAD, :], 0.0)
        return out + bias_ref[:][None]

    xp0 = jnp.dot(x, w0_ref[:], preferred_element_type=f32)
    h1 = jnp.maximum(gat_layer(xp0, a0_ref, b0_ref), 0.0)
    xp1 = jnp.dot(h1.reshape(G * NPAD, H), w1_ref[:],
                  preferred_element_type=f32)
    out = jnp.maximum(gat_layer(xp1, a1_ref, b1_ref), 0.0)
    outh_ref[:] = out[:, 0, :]
    outv_ref[:] = out[:, 1 + NOBJ:1 + NOBJ + NVAL, :]


def _att_mat(att):
    return (jnp.eye(NHEADS, dtype=jnp.float32)[:, None, :]
            * att[:, :, None]).reshape(H, NHEADS)


@jax.jit
def kernel(head_node, objective_nodes, value_nodes, edge_indices,
           W_head, b_head, W_obj, b_obj, W_val, b_val,
           W0, att_src0, att_dst0, bias0,
           W1, att_src1, att_dst1, bias1):
    b = head_node.shape[0]
    f32 = jnp.float32
    obj2 = objective_nodes.reshape(b * NOBJ, 2)
    val2 = value_nodes.reshape(b * NVAL, 5)
    a0 = jnp.concatenate([_att_mat(att_src0), _att_mat(att_dst0)], axis=1)
    a1 = jnp.concatenate([_att_mat(att_src1), _att_mat(att_dst1)], axis=1)

    full = lambda *shape: pl.BlockSpec(shape, lambda i: tuple(0 for _ in shape))
    outh, outv = pl.pallas_call(
        _kernel,
        grid=(b // G,),
        in_specs=[
            pl.BlockSpec((G, 2), lambda i: (i, 0)),
            pl.BlockSpec((G * NOBJ, 2), lambda i: (i, 0)),
            pl.BlockSpec((G * NVAL, 5), lambda i: (i, 0)),
            pl.BlockSpec((G, 2, E), lambda i: (i, 0, 0)),
            full(2, H), full(1, H), full(2, H), full(1, H),
            full(5, H), full(1, H),
            full(H, H), full(H, 2 * NHEADS), full(1, H),
            full(H, H), full(H, 2 * NHEADS), full(1, H),
        ],
        out_specs=(pl.BlockSpec((G, H), lambda i: (i, 0)),
                   pl.BlockSpec((G, NVAL, H), lambda i: (i, 0, 0))),
        out_shape=(jax.ShapeDtypeStruct((b, H), f32),
                   jax.ShapeDtypeStruct((b, NVAL, H), f32)),
    )(head_node, obj2, val2, edge_indices,
      W_head, b_head[None, :], W_obj, b_obj[None, :], W_val, b_val[None, :],
      W0, a0, bias0[None, :], W1, a1, bias1[None, :])
    return outh, outv
